# baseline (speedup 1.0000x reference)
import functools

import jax
import jax.numpy as jnp
import numpy as np
from jax import lax
from jax.experimental import pallas as pl
from jax.experimental.pallas import tpu as pltpu
from jax.experimental.pallas import tpu_sc as plsc

B = 16384
NUMF = 13
L = 20
NFIELDS = 26
VOCAB = 100000
RC = 64

NC = 2
NS = 16
NW = NC * NS
BPW = B // NW
CB = 32
NCHUNK = BPW // CB
ROWS = CB * NFIELDS
NDMA = 8
RPD = ROWS // NDMA


def _sc_emb_sum(flat_table, flat_idx):
    mesh = plsc.VectorSubcoreMesh(core_axis_name="c", subcore_axis_name="s")

    @functools.partial(
        pl.kernel,
        mesh=mesh,
        out_type=jax.ShapeDtypeStruct((B, RC), jnp.float32),
        scratch_types=[
            pltpu.VMEM((NDMA, RPD), jnp.int32),
            pltpu.VMEM((ROWS, RC), jnp.float32),
            pltpu.VMEM((CB, RC), jnp.float32),
            pltpu.SemaphoreType.DMA,
        ],
        compiler_params=pltpu.CompilerParams(use_tc_tiling_on_sc=False),
    )
    def k(table_hbm, idx_hbm, out_hbm, idx_v, rows_v, acc_v, sem):
        wid = lax.axis_index("s") * NC + lax.axis_index("c")
        idx_row0 = wid * (NCHUNK * NDMA)
        out_row0 = wid * BPW

        def chunk_body(c, _):
            pltpu.sync_copy(idx_hbm.at[pl.ds(idx_row0 + c * NDMA, NDMA)],
                            idx_v)
            cps = []
            for j in range(NDMA):
                cps.append(pltpu.async_copy(
                    table_hbm.at[idx_v.at[j]],
                    rows_v.at[pl.ds(j * RPD, RPD)],
                    sem))
            for cp in cps:
                cp.wait()

            def red_body(b, _):
                r0 = b * NFIELDS
                for kk in range(RC // 16):
                    v = rows_v[r0, pl.ds(kk * 16, 16)]
                    for f in range(1, NFIELDS):
                        v = v + rows_v[r0 + f, pl.ds(kk * 16, 16)]
                    acc_v[b, pl.ds(kk * 16, 16)] = v
                return 0

            lax.fori_loop(0, CB, red_body, 0)
            pltpu.sync_copy(acc_v, out_hbm.at[pl.ds(out_row0 + c * CB, CB)])
            return 0

        lax.fori_loop(0, NCHUNK, chunk_body, 0)

    return k(flat_table, flat_idx)


def _tc_stats(x2):
    def body(x_ref, s_ref, q_ref):
        x = x_ref[...]
        s_ref[...] = jnp.sum(x, axis=0, keepdims=True)
        q_ref[...] = jnp.sum(x * x, axis=0, keepdims=True)

    return pl.pallas_call(
        body,
        out_shape=(jax.ShapeDtypeStruct((1, NUMF * L), jnp.float32),
                   jax.ShapeDtypeStruct((1, NUMF * L), jnp.float32)),
    )(x2)


def _tc_main(x2, emb, m1, m2, bias):
    BB = 1024
    grid = (B // BB,)

    def body(x_ref, e_ref, m1_ref, m2_ref, b_ref, o_ref):
        o_ref[...] = (
            jnp.dot(x_ref[...], m1_ref[...], preferred_element_type=jnp.float32)
            + jnp.dot(e_ref[...], m2_ref[...], preferred_element_type=jnp.float32)
            + b_ref[...])

    return pl.pallas_call(
        body,
        grid=grid,
        in_specs=[
            pl.BlockSpec((BB, NUMF * L), lambda i: (i, 0)),
            pl.BlockSpec((BB, RC), lambda i: (i, 0)),
            pl.BlockSpec((NUMF * L, RC * L), lambda i: (0, 0)),
            pl.BlockSpec((RC, RC * L), lambda i: (0, 0)),
            pl.BlockSpec((1, RC * L), lambda i: (0, 0)),
        ],
        out_specs=pl.BlockSpec((BB, RC * L), lambda i: (i, 0)),
        out_shape=jax.ShapeDtypeStruct((B, RC * L), jnp.float32),
    )(x2, emb, m1, m2, bias)


def kernel(num_feat, emb_feat, bn_gamma, bn_beta, conv_w, conv_b, emb_tables):
    flat_table = emb_tables.reshape(NFIELDS * VOCAB, RC)
    flat_idx = (emb_feat + (jnp.arange(NFIELDS, dtype=jnp.int32) * VOCAB)[None, :]
                ).reshape(B * NFIELDS // RPD, RPD)
    emb_sum = _sc_emb_sum(flat_table, flat_idx)

    x2 = num_feat.reshape(B, NUMF * L)
    sums, sumsq = _tc_stats(x2)

    n = float(B * L)
    s_c = sums.reshape(NUMF, L).sum(axis=1)
    q_c = sumsq.reshape(NUMF, L).sum(axis=1)
    mean = s_c / n
    var = q_c / n - mean * mean
    scale = bn_gamma / jnp.sqrt(var + 1e-5)
    shift = bn_beta - mean * scale
    w = conv_w[:, :, 0]
    weff = w * scale[None, :]
    bias_o = w @ shift + conv_b
    eye_l = jnp.eye(L, dtype=jnp.float32)
    m1 = (weff.T[:, None, :, None] * eye_l[None, :, None, :]).reshape(
        NUMF * L, RC * L)
    m2 = jnp.asarray(np.kron(np.eye(RC, dtype=np.float32),
                             np.ones((1, L), dtype=np.float32)))
    bias = jnp.repeat(bias_o, L)[None, :]

    out2 = _tc_main(x2, emb_sum, m1, m2, bias)
    return out2.reshape(B, RC, L)


# speedup vs baseline: 1.1735x; 1.1735x over previous
import functools

import jax
import jax.numpy as jnp
from jax import lax
from jax.experimental import pallas as pl
from jax.experimental.pallas import tpu as pltpu
from jax.experimental.pallas import tpu_sc as plsc

B = 16384
NUMF = 13
L = 20
NFIELDS = 26
VOCAB = 100000
RC = 64

NC = 2
NS = 16
NW = NC * NS
BPW = B // NW
CB = 32
NCHUNK = BPW // CB


def _sc_emb_sum_t(flat_table, flat_idx):
    mesh = plsc.VectorSubcoreMesh(core_axis_name="c", subcore_axis_name="s")

    @functools.partial(
        pl.kernel,
        mesh=mesh,
        out_type=jax.ShapeDtypeStruct((RC, B), jnp.float32),
        scratch_types=[
            pltpu.VMEM((NFIELDS, CB), jnp.int32),
            pltpu.VMEM((NFIELDS, CB), jnp.int32),
            pltpu.VMEM((NFIELDS * CB, RC), jnp.float32),
            pltpu.VMEM((NFIELDS * CB, RC), jnp.float32),
            pltpu.VMEM((RC, CB), jnp.float32),
            pltpu.SemaphoreType.DMA,
            pltpu.SemaphoreType.DMA,
        ],
        compiler_params=pltpu.CompilerParams(use_tc_tiling_on_sc=False,
                                             needs_layout_passes=False),
    )
    def k(table_hbm, idx_hbm, out_hbm, idx0, idx1, rows0, rows1, acc,
          sem0, sem1):
        wid = lax.axis_index("s") * NC + lax.axis_index("c")
        b00 = wid * BPW
        idxs = (idx0, idx1)
        rowss = (rows0, rows1)
        sems = (sem0, sem1)

        def stage_and_fire(c, ib, rb, sb):
            pltpu.sync_copy(idx_hbm.at[:, pl.ds(b00 + c * CB, CB)], ib)
            for f in range(NFIELDS):
                pltpu.async_copy(table_hbm.at[ib.at[f]],
                                 rb.at[pl.ds(f * CB, CB)], sb)

        def drain(ib, rb, sb):
            for f in range(NFIELDS):
                pltpu.make_async_copy(table_hbm.at[ib.at[f]],
                                      rb.at[pl.ds(f * CB, CB)], sb).wait()

        stage_and_fire(0, idx0, rows0, sem0)
        o_lanes = [lax.iota(jnp.int32, 16) + kk * 16
                   for kk in range(RC // 16)]

        def pair_body(i, _):
            for p in range(2):
                c = 2 * i + p
                nc = c + 1
                ib, rb, sb = idxs[p], rowss[p], sems[p]
                ob, rob, sob = idxs[1 - p], rowss[1 - p], sems[1 - p]

                @pl.when(nc < NCHUNK)
                def _():
                    stage_and_fire(nc, ob, rob, sob)

                drain(ib, rb, sb)

                def red_body(b, _):
                    b_lane = jnp.full((16,), b, dtype=jnp.int32)
                    for kk in range(RC // 16):
                        vs = [rb[f * CB + b, pl.ds(kk * 16, 16)]
                              for f in range(NFIELDS)]
                        while len(vs) > 1:
                            nxt = [vs[j] + vs[j + 1]
                                   for j in range(0, len(vs) - 1, 2)]
                            if len(vs) % 2:
                                nxt.append(vs[-1])
                            vs = nxt
                        plsc.store_scatter(acc, [o_lanes[kk], b_lane], vs[0])
                    return 0

                lax.fori_loop(0, CB, red_body, 0)
                pltpu.sync_copy(acc, out_hbm.at[:, pl.ds(b00 + c * CB, CB)])
            return 0

        lax.fori_loop(0, NCHUNK // 2, pair_body, 0)

    return k(flat_table, flat_idx)


def _tc_stats(xt):
    def body(x_ref, s_ref):
        x = x_ref[...]
        s_ref[0, :] = jnp.sum(jnp.sum(x, axis=1), axis=1)
        s_ref[1, :] = jnp.sum(jnp.sum(x * x, axis=1), axis=1)

    return pl.pallas_call(
        body,
        out_shape=jax.ShapeDtypeStruct((2, NUMF), jnp.float32),
    )(xt)


def _tc_main(xt, emb_t, weff, bias_c):
    BBK = 2048
    grid = (B // BBK,)

    def body(w_ref, b_ref, x_ref, e_ref, o_ref):
        w = w_ref[...]
        e = e_ref[...] + b_ref[...]
        for l in range(L):
            o_ref[l] = jnp.dot(w, x_ref[:, l, :],
                               preferred_element_type=jnp.float32) + e

    return pl.pallas_call(
        body,
        grid=grid,
        in_specs=[
            pl.BlockSpec((RC, NUMF), lambda j: (0, 0)),
            pl.BlockSpec((RC, 1), lambda j: (0, 0)),
            pl.BlockSpec((NUMF, L, BBK), lambda j: (0, 0, j)),
            pl.BlockSpec((RC, BBK), lambda j: (0, j)),
        ],
        out_specs=pl.BlockSpec((L, RC, BBK), lambda j: (0, 0, j)),
        out_shape=jax.ShapeDtypeStruct((L, RC, B), jnp.float32),
    )(weff, bias_c, xt, emb_t)


def kernel(num_feat, emb_feat, bn_gamma, bn_beta, conv_w, conv_b, emb_tables):
    xt = jnp.transpose(num_feat, (1, 2, 0))
    femb = jnp.transpose(emb_feat, (1, 0))

    flat_table = emb_tables.reshape(NFIELDS * VOCAB, RC)
    flat_idx = femb + (jnp.arange(NFIELDS, dtype=jnp.int32) * VOCAB)[:, None]
    emb_t = _sc_emb_sum_t(flat_table, flat_idx)

    stats = _tc_stats(xt)

    n = float(B * L)
    mean = stats[0] / n
    var = stats[1] / n - mean * mean
    scale = bn_gamma / jnp.sqrt(var + 1e-5)
    shift = bn_beta - mean * scale
    w = conv_w[:, :, 0]
    weff = w * scale[None, :]
    bias_c = (w @ shift + conv_b)[:, None]

    out_t = _tc_main(xt, emb_t, weff, bias_c)
    return jnp.transpose(out_t, (2, 1, 0))
